# baseline (speedup 1.0000x reference)
import functools

import jax
import jax.numpy as jnp
from jax import lax
from jax.experimental import pallas as pl
from jax.experimental.pallas import tpu as pltpu
from jax.experimental.pallas import tpu_sc as plsc

N = 10000
E_IN = 160000
D = 128
DA = 4
DF = 8
MID = D * DA
AVG = 16.0

NW = 32
CH = 128
NCH = 40
EPW = CH * NCH
E_PAD = EPW * NW
RPT = N // 16

_mesh = plsc.VectorSubcoreMesh(core_axis_name="c", subcore_axis_name="s")


def _node_body(x_ref, ws_ref, wu_ref, w1a_ref, w1b_ref, ts_ref, tr_ref):
    x = x_ref[...]
    ns = jnp.dot(x, ws_ref[...], preferred_element_type=jnp.float32)
    ts_ref[:, :D] = jnp.dot(ns, w1a_ref[...], preferred_element_type=jnp.float32)
    ts_ref[:, D:] = jnp.dot(x, wu_ref[...], preferred_element_type=jnp.float32)
    tr_ref[...] = jnp.dot(ns, w1b_ref[...], preferred_element_type=jnp.float32)


def _node_precompute(x, ws, wu, w1a, w1b):
    blk = 1000
    grid = N // blk
    full = lambda i: (0, 0)
    return pl.pallas_call(
        _node_body,
        grid=(grid,),
        in_specs=[
            pl.BlockSpec((blk, D), lambda i: (i, 0)),
            pl.BlockSpec((D, D), full),
            pl.BlockSpec((D, D), full),
            pl.BlockSpec((D, D), full),
            pl.BlockSpec((D, D), full),
        ],
        out_specs=[
            pl.BlockSpec((blk, 2 * D), lambda i: (i, 0)),
            pl.BlockSpec((blk, D), lambda i: (i, 0)),
        ],
        out_shape=[
            jax.ShapeDtypeStruct((N, 2 * D), jnp.float32),
            jax.ShapeDtypeStruct((N, D), jnp.float32),
        ],
    )(x, ws, wu, w1a, w1b)


@functools.partial(
    pl.kernel,
    mesh=_mesh,
    out_type=[
        jax.ShapeDtypeStruct((E_PAD, D), jnp.float32),
        jax.ShapeDtypeStruct((E_PAD, D), jnp.float32),
    ],
    scratch_types=[
        pltpu.VMEM((EPW,), jnp.int32),
        pltpu.VMEM((EPW,), jnp.int32),
        pltpu.VMEM((2, CH, D), jnp.float32),
        pltpu.VMEM((2, CH, D), jnp.float32),
        pltpu.SemaphoreType.DMA,
        pltpu.SemaphoreType.DMA,
        pltpu.SemaphoreType.DMA,
        pltpu.SemaphoreType.DMA,
        pltpu.SemaphoreType.DMA,
        pltpu.SemaphoreType.DMA,
        pltpu.SemaphoreType.DMA,
        pltpu.SemaphoreType.DMA,
    ],
)
def _gather(ts_hbm, tr_hbm, snd_hbm, rcv_hbm, gs_hbm, gr_hbm,
            idx_s, idx_r, rows_s, rows_r,
            sg0, sg1, sr0, sr1, ws0, ws1, wr0, wr1):
    wid = lax.axis_index("c") * 16 + lax.axis_index("s")
    base = wid * EPW
    pltpu.sync_copy(snd_hbm.at[pl.ds(base, EPW)], idx_s)
    pltpu.sync_copy(rcv_hbm.at[pl.ds(base, EPW)], idx_r)
    sg = (sg0, sg1)
    sr = (sr0, sr1)
    ws = (ws0, ws1)
    wr = (wr0, wr1)

    def body(t, carry):
        g_h, w_h = [], []
        for b in range(2):
            off = (2 * t + b) * CH
            g_h.append(pltpu.async_copy(
                ts_hbm.at[idx_s.at[pl.ds(off, CH)]], rows_s.at[b], sg[b]))
            g_h.append(pltpu.async_copy(
                tr_hbm.at[idx_r.at[pl.ds(off, CH)]], rows_r.at[b], sr[b]))
        for b in range(2):
            off = (2 * t + b) * CH
            g_h[2 * b].wait()
            w_h.append(pltpu.async_copy(
                rows_s.at[b], gs_hbm.at[pl.ds(base + off, CH)], ws[b]))
            g_h[2 * b + 1].wait()
            w_h.append(pltpu.async_copy(
                rows_r.at[b], gr_hbm.at[pl.ds(base + off, CH)], wr[b]))
        for h in w_h:
            h.wait()
        return carry

    lax.fori_loop(0, NCH // 2, body, 0)


def _edge_body(gs_ref, gr_ref, ef_ref, ln_ref, ea_ref,
               w1c_ref, w1d_ref, b1_ref, w2_ref, b2_ref, w3g_ref, wog_ref,
               c_ref):
    a_s = gs_ref[:, :D].astype(jnp.float32)
    nf_s = gs_ref[:, D:].astype(jnp.float32)
    pre = a_s + gr_ref[...] + b1_ref[...]
    pre = pre + jnp.dot(ef_ref[...], w1c_ref[...], preferred_element_type=jnp.float32)
    pre = pre + ln_ref[...] * w1d_ref[...]
    h1 = pre * jax.nn.sigmoid(pre)
    pre2 = jnp.dot(h1.astype(jnp.bfloat16), w2_ref[...],
                   preferred_element_type=jnp.float32) + b2_ref[...]
    h2 = pre2 * jax.nn.sigmoid(pre2)
    tpw = jnp.dot(h2.astype(jnp.bfloat16), w3g_ref[...],
                  preferred_element_type=jnp.float32)
    acc = None
    for a in range(DA):
        m = nf_s * tpw[:, a * D:(a + 1) * D] * ea_ref[:, a:a + 1]
        part = jnp.dot(m.astype(jnp.bfloat16), wog_ref[a],
                       preferred_element_type=jnp.float32)
        acc = part if acc is None else acc + part
    c_ref[...] = acc


def _edge_compute(gs, gr, ef, ln, ea, w1c, w1d, b1r, w2, b2r, w3g, wog):
    blk = 1024
    grid = E_PAD // blk
    full = lambda i: (0, 0)
    return pl.pallas_call(
        _edge_body,
        grid=(grid,),
        in_specs=[
            pl.BlockSpec((blk, 2 * D), lambda i: (i, 0)),
            pl.BlockSpec((blk, D), lambda i: (i, 0)),
            pl.BlockSpec((blk, DF), lambda i: (i, 0)),
            pl.BlockSpec((blk, 1), lambda i: (i, 0)),
            pl.BlockSpec((blk, DA), lambda i: (i, 0)),
            pl.BlockSpec((DF, D), full),
            pl.BlockSpec((1, D), full),
            pl.BlockSpec((1, D), full),
            pl.BlockSpec((D, D), full),
            pl.BlockSpec((1, D), full),
            pl.BlockSpec((D, MID), full),
            pl.BlockSpec((DA, D, D), lambda i: (0, 0, 0)),
        ],
        out_specs=pl.BlockSpec((blk, D), lambda i: (i, 0)),
        out_shape=jax.ShapeDtypeStruct((E_PAD, D), jnp.float32),
    )(gs, gr, ef, ln, ea, w1c, w1d, b1r, w2, b2r, w3g, wog)


@functools.partial(
    pl.kernel,
    mesh=_mesh,
    out_type=jax.ShapeDtypeStruct((2, 16, RPT, D), jnp.float32),
    scratch_types=[
        pltpu.VMEM((NCH, CH), jnp.int32),
        pltpu.VMEM((2, CH, D), jnp.float32),
        pltpu.VMEM_SHARED((N, D), jnp.float32),
        pltpu.SemaphoreType.DMA,
        pltpu.SemaphoreType.DMA,
    ],
)
def _scatter(c_hbm, rcv3_hbm, zeros_hbm, out_hbm, idx_v, rows_v, accum,
             sl0, sl1):
    c = lax.axis_index("c")
    s = lax.axis_index("s")
    wid = c * 16 + s
    base = wid * EPW
    pltpu.sync_copy(zeros_hbm, accum.at[pl.ds(s * RPT, RPT)])
    pltpu.sync_copy(rcv3_hbm.at[wid], idx_v)
    plsc.subcore_barrier()
    sl = (sl0, sl1)

    def body(t, carry):
        h = []
        for b in range(2):
            j = 2 * t + b
            h.append(pltpu.async_copy(
                c_hbm.at[pl.ds(base + j * CH, CH)], rows_v.at[b], sl[b]))
        for b in range(2):
            j = 2 * t + b
            h[b].wait()
            pltpu.sync_copy(rows_v.at[b], accum.at[idx_v.at[j]], add=True)
        return carry

    lax.fori_loop(0, NCH // 2, body, 0)
    plsc.subcore_barrier()
    pltpu.sync_copy(accum.at[pl.ds(s * RPT, RPT)], out_hbm.at[c, s])


def _add_body(p0_ref, p1_ref, o_ref):
    o_ref[...] = p0_ref[...] + p1_ref[...]


def _final_add(p0, p1):
    blk = 1000
    return pl.pallas_call(
        _add_body,
        grid=(N // blk,),
        in_specs=[
            pl.BlockSpec((blk, D), lambda i: (i, 0)),
            pl.BlockSpec((blk, D), lambda i: (i, 0)),
        ],
        out_specs=pl.BlockSpec((blk, D), lambda i: (i, 0)),
        out_shape=jax.ShapeDtypeStruct((N, D), jnp.float32),
    )(p0, p1)


def kernel(node_feats, edge_attrs, edge_feats, lengths, edge_index,
           W_scalar, W_up, W1, b1, W2, b2, W3, W_out):
    pad = E_PAD - E_IN
    snd = jnp.pad(edge_index[0], (0, pad))
    rcv = jnp.pad(edge_index[1], (0, pad))
    ea = jnp.pad(edge_attrs, ((0, pad), (0, 0)))
    ef = jnp.pad(edge_feats, ((0, pad), (0, 0)))
    ln = jnp.pad(lengths, ((0, pad), (0, 0)))

    w1a = W1[:D]
    w1b = W1[D:2 * D]
    w1c = W1[2 * D:2 * D + DF]
    w1d = W1[2 * D + DF:2 * D + DF + 1]
    b1r = b1.reshape(1, D)
    b2r = b2.reshape(1, D)
    w3g = W3.reshape(D, D, DA).transpose(0, 2, 1).reshape(D, MID)
    wog = W_out.reshape(D, DA, D).transpose(1, 0, 2) / AVG

    ts, tr = _node_precompute(node_feats, W_scalar, W_up, w1a, w1b)
    ts_p = lax.bitcast_convert_type(
        ts.astype(jnp.bfloat16).reshape(N, D, 2), jnp.float32)
    gs_p, gr = _gather(ts_p, tr, snd, rcv)
    gs = lax.bitcast_convert_type(gs_p, jnp.bfloat16).reshape(E_PAD, 2 * D)
    cmsg = _edge_compute(gs, gr, ef, ln, ea, w1c, w1d, b1r,
                         W2.astype(jnp.bfloat16), b2r,
                         w3g.astype(jnp.bfloat16), wog.astype(jnp.bfloat16))
    partials = _scatter(cmsg, rcv.reshape(NW, NCH, CH),
                        jnp.zeros((RPT, D), jnp.float32)).reshape(2, N, D)
    out = _final_add(partials[0], partials[1])
    return out.reshape(N, D, 1)


# speedup vs baseline: 1.0345x; 1.0345x over previous
import functools

import jax
import jax.numpy as jnp
from jax import lax
from jax.experimental import pallas as pl
from jax.experimental.pallas import tpu as pltpu
from jax.experimental.pallas import tpu_sc as plsc

N = 10000
E_IN = 160000
D = 128
DA = 4
DF = 8
MID = D * DA
AVG = 16.0

NW = 32
CH = 128
NCH = 40
EPW = CH * NCH
E_PAD = EPW * NW
RPT = N // 16

_mesh = plsc.VectorSubcoreMesh(core_axis_name="c", subcore_axis_name="s")


def _node_body(x_ref, ws_ref, wu_ref, w1a_ref, w1b_ref, ts_ref, tr_ref):
    x = x_ref[...]
    ns = jnp.dot(x, ws_ref[...], preferred_element_type=jnp.float32)
    ts_ref[:, :D] = jnp.dot(ns, w1a_ref[...], preferred_element_type=jnp.float32)
    ts_ref[:, D:] = jnp.dot(x, wu_ref[...], preferred_element_type=jnp.float32)
    tr_ref[...] = jnp.dot(ns, w1b_ref[...], preferred_element_type=jnp.float32)


def _node_precompute(x, ws, wu, w1a, w1b):
    blk = 1000
    grid = N // blk
    full = lambda i: (0, 0)
    return pl.pallas_call(
        _node_body,
        grid=(grid,),
        in_specs=[
            pl.BlockSpec((blk, D), lambda i: (i, 0)),
            pl.BlockSpec((D, D), full),
            pl.BlockSpec((D, D), full),
            pl.BlockSpec((D, D), full),
            pl.BlockSpec((D, D), full),
        ],
        out_specs=[
            pl.BlockSpec((blk, 2 * D), lambda i: (i, 0)),
            pl.BlockSpec((blk, D), lambda i: (i, 0)),
        ],
        out_shape=[
            jax.ShapeDtypeStruct((N, 2 * D), jnp.float32),
            jax.ShapeDtypeStruct((N, D), jnp.float32),
        ],
    )(x, ws, wu, w1a, w1b)


@functools.partial(
    pl.kernel,
    mesh=_mesh,
    out_type=[
        jax.ShapeDtypeStruct((E_PAD, D), jnp.float32),
        jax.ShapeDtypeStruct((E_PAD, D), jnp.float32),
    ],
    scratch_types=[
        pltpu.VMEM((EPW,), jnp.int32),
        pltpu.VMEM((EPW,), jnp.int32),
        pltpu.VMEM((2, CH, D), jnp.float32),
        pltpu.VMEM((2, CH, D), jnp.float32),
        pltpu.SemaphoreType.DMA,
        pltpu.SemaphoreType.DMA,
        pltpu.SemaphoreType.DMA,
        pltpu.SemaphoreType.DMA,
        pltpu.SemaphoreType.DMA,
        pltpu.SemaphoreType.DMA,
        pltpu.SemaphoreType.DMA,
        pltpu.SemaphoreType.DMA,
    ],
)
def _gather(ts_hbm, tr_hbm, snd_hbm, rcv_hbm, gs_hbm, gr_hbm,
            idx_s, idx_r, rows_s, rows_r,
            sg0, sg1, sr0, sr1, ws0, ws1, wr0, wr1):
    wid = lax.axis_index("c") * 16 + lax.axis_index("s")
    base = wid * EPW
    pltpu.sync_copy(snd_hbm.at[pl.ds(base, EPW)], idx_s)
    pltpu.sync_copy(rcv_hbm.at[pl.ds(base, EPW)], idx_r)
    sg = (sg0, sg1)
    sr = (sr0, sr1)
    ws = (ws0, ws1)
    wr = (wr0, wr1)

    def body(t, carry):
        g_h, w_h = [], []
        for b in range(2):
            off = (2 * t + b) * CH
            g_h.append(pltpu.async_copy(
                ts_hbm.at[idx_s.at[pl.ds(off, CH)]], rows_s.at[b], sg[b]))
            g_h.append(pltpu.async_copy(
                tr_hbm.at[idx_r.at[pl.ds(off, CH)]], rows_r.at[b], sr[b]))
        for b in range(2):
            off = (2 * t + b) * CH
            g_h[2 * b].wait()
            w_h.append(pltpu.async_copy(
                rows_s.at[b], gs_hbm.at[pl.ds(base + off, CH)], ws[b]))
            g_h[2 * b + 1].wait()
            w_h.append(pltpu.async_copy(
                rows_r.at[b], gr_hbm.at[pl.ds(base + off, CH)], wr[b]))
        for h in w_h:
            h.wait()
        return carry

    lax.fori_loop(0, NCH // 2, body, 0)


def _edge_body(gs_ref, gr_ref, ef_ref, ln_ref, ea_ref,
               w1c_ref, w1d_ref, b1_ref, w2_ref, b2_ref, w3g_ref, wog_ref,
               c_ref):
    a_s = gs_ref[:, :D].astype(jnp.float32)
    nf_s = gs_ref[:, D:].astype(jnp.float32)
    pre = a_s + gr_ref[...] + b1_ref[...]
    pre = pre + jnp.dot(ef_ref[...], w1c_ref[...], preferred_element_type=jnp.float32)
    pre = pre + ln_ref[...] * w1d_ref[...]
    h1 = pre * jax.nn.sigmoid(pre)
    pre2 = jnp.dot(h1, w2_ref[...], preferred_element_type=jnp.float32) + b2_ref[...]
    h2 = pre2 * jax.nn.sigmoid(pre2)
    tpw = jnp.dot(h2, w3g_ref[...], preferred_element_type=jnp.float32)
    acc = None
    for a in range(DA):
        m = nf_s * tpw[:, a * D:(a + 1) * D] * ea_ref[:, a:a + 1]
        part = jnp.dot(m, wog_ref[a], preferred_element_type=jnp.float32)
        acc = part if acc is None else acc + part
    c_ref[...] = acc


def _edge_compute(gs, gr, ef, ln, ea, w1c, w1d, b1r, w2, b2r, w3g, wog):
    blk = 1024
    grid = E_PAD // blk
    full = lambda i: (0, 0)
    return pl.pallas_call(
        _edge_body,
        grid=(grid,),
        in_specs=[
            pl.BlockSpec((blk, 2 * D), lambda i: (i, 0)),
            pl.BlockSpec((blk, D), lambda i: (i, 0)),
            pl.BlockSpec((blk, DF), lambda i: (i, 0)),
            pl.BlockSpec((blk, 1), lambda i: (i, 0)),
            pl.BlockSpec((blk, DA), lambda i: (i, 0)),
            pl.BlockSpec((DF, D), full),
            pl.BlockSpec((1, D), full),
            pl.BlockSpec((1, D), full),
            pl.BlockSpec((D, D), full),
            pl.BlockSpec((1, D), full),
            pl.BlockSpec((D, MID), full),
            pl.BlockSpec((DA, D, D), lambda i: (0, 0, 0)),
        ],
        out_specs=pl.BlockSpec((blk, D), lambda i: (i, 0)),
        out_shape=jax.ShapeDtypeStruct((E_PAD, D), jnp.float32),
    )(gs, gr, ef, ln, ea, w1c, w1d, b1r, w2, b2r, w3g, wog)


@functools.partial(
    pl.kernel,
    mesh=_mesh,
    out_type=jax.ShapeDtypeStruct((2, 16, RPT, D), jnp.float32),
    scratch_types=[
        pltpu.VMEM((NCH, CH), jnp.int32),
        pltpu.VMEM((2, CH, D), jnp.float32),
        pltpu.VMEM_SHARED((N, D), jnp.float32),
        pltpu.SemaphoreType.DMA,
        pltpu.SemaphoreType.DMA,
    ],
)
def _scatter(c_hbm, rcv3_hbm, zeros_hbm, out_hbm, idx_v, rows_v, accum,
             sl0, sl1):
    c = lax.axis_index("c")
    s = lax.axis_index("s")
    wid = c * 16 + s
    base = wid * EPW
    pltpu.sync_copy(zeros_hbm, accum.at[pl.ds(s * RPT, RPT)])
    pltpu.sync_copy(rcv3_hbm.at[wid], idx_v)
    plsc.subcore_barrier()
    sl = (sl0, sl1)

    def body(t, carry):
        h = []
        for b in range(2):
            j = 2 * t + b
            h.append(pltpu.async_copy(
                c_hbm.at[pl.ds(base + j * CH, CH)], rows_v.at[b], sl[b]))
        for b in range(2):
            j = 2 * t + b
            h[b].wait()
            pltpu.sync_copy(rows_v.at[b], accum.at[idx_v.at[j]], add=True)
        return carry

    lax.fori_loop(0, NCH // 2, body, 0)
    plsc.subcore_barrier()
    pltpu.sync_copy(accum.at[pl.ds(s * RPT, RPT)], out_hbm.at[c, s])


def _add_body(p0_ref, p1_ref, o_ref):
    o_ref[...] = p0_ref[...] + p1_ref[...]


def _final_add(p0, p1):
    blk = 1000
    return pl.pallas_call(
        _add_body,
        grid=(N // blk,),
        in_specs=[
            pl.BlockSpec((blk, D), lambda i: (i, 0)),
            pl.BlockSpec((blk, D), lambda i: (i, 0)),
        ],
        out_specs=pl.BlockSpec((blk, D), lambda i: (i, 0)),
        out_shape=jax.ShapeDtypeStruct((N, D), jnp.float32),
    )(p0, p1)


def kernel(node_feats, edge_attrs, edge_feats, lengths, edge_index,
           W_scalar, W_up, W1, b1, W2, b2, W3, W_out):
    pad = E_PAD - E_IN
    snd = jnp.pad(edge_index[0], (0, pad))
    rcv = jnp.pad(edge_index[1], (0, pad))
    ea = jnp.pad(edge_attrs, ((0, pad), (0, 0)))
    ef = jnp.pad(edge_feats, ((0, pad), (0, 0)))
    ln = jnp.pad(lengths, ((0, pad), (0, 0)))

    w1a = W1[:D]
    w1b = W1[D:2 * D]
    w1c = W1[2 * D:2 * D + DF]
    w1d = W1[2 * D + DF:2 * D + DF + 1]
    b1r = b1.reshape(1, D)
    b2r = b2.reshape(1, D)
    w3g = W3.reshape(D, D, DA).transpose(0, 2, 1).reshape(D, MID)
    wog = W_out.reshape(D, DA, D).transpose(1, 0, 2) / AVG

    ts, tr = _node_precompute(node_feats, W_scalar, W_up, w1a, w1b)
    ts_p = lax.bitcast_convert_type(
        ts.astype(jnp.bfloat16).reshape(N, D, 2), jnp.float32)
    gs_p, gr = _gather(ts_p, tr, snd, rcv)
    gs = lax.bitcast_convert_type(gs_p, jnp.bfloat16).reshape(E_PAD, 2 * D)
    cmsg = _edge_compute(gs, gr, ef, ln, ea, w1c, w1d, b1r, W2, b2r, w3g, wog)
    partials = _scatter(cmsg, rcv.reshape(NW, NCH, CH),
                        jnp.zeros((RPT, D), jnp.float32)).reshape(2, N, D)
    out = _final_add(partials[0], partials[1])
    return out.reshape(N, D, 1)


# speedup vs baseline: 1.6902x; 1.6338x over previous
import functools

import jax
import jax.numpy as jnp
from jax import lax
from jax.experimental import pallas as pl
from jax.experimental.pallas import tpu as pltpu
from jax.experimental.pallas import tpu_sc as plsc

N = 10000
E_IN = 160000
D = 128
DA = 4
DF = 8
MID = D * DA
AVG = 16.0

NW = 32
CH = 128
NCH = 40
EPW = CH * NCH
E_PAD = EPW * NW
RPT = N // 16

_mesh = plsc.VectorSubcoreMesh(core_axis_name="c", subcore_axis_name="s")


def _node_body(x_ref, ws_ref, wu_ref, w1a_ref, w1b_ref, ts_ref, tr_ref):
    x = x_ref[...]
    ns = jnp.dot(x, ws_ref[...], preferred_element_type=jnp.float32)
    ts_ref[:, :D] = jnp.dot(ns, w1a_ref[...], preferred_element_type=jnp.float32)
    ts_ref[:, D:] = jnp.dot(x, wu_ref[...], preferred_element_type=jnp.float32)
    tr_ref[...] = jnp.dot(ns, w1b_ref[...], preferred_element_type=jnp.float32)


def _node_precompute(x, ws, wu, w1a, w1b):
    blk = 1000
    grid = N // blk
    full = lambda i: (0, 0)
    return pl.pallas_call(
        _node_body,
        grid=(grid,),
        in_specs=[
            pl.BlockSpec((blk, D), lambda i: (i, 0)),
            pl.BlockSpec((D, D), full),
            pl.BlockSpec((D, D), full),
            pl.BlockSpec((D, D), full),
            pl.BlockSpec((D, D), full),
        ],
        out_specs=[
            pl.BlockSpec((blk, 2 * D), lambda i: (i, 0)),
            pl.BlockSpec((blk, D), lambda i: (i, 0)),
        ],
        out_shape=[
            jax.ShapeDtypeStruct((N, 2 * D), jnp.float32),
            jax.ShapeDtypeStruct((N, D), jnp.float32),
        ],
    )(x, ws, wu, w1a, w1b)


@functools.partial(
    pl.kernel,
    mesh=_mesh,
    out_type=[
        jax.ShapeDtypeStruct((E_PAD, 2 * D), jnp.float32),
        jax.ShapeDtypeStruct((E_PAD, D), jnp.float32),
    ],
    scratch_types=[
        pltpu.VMEM((EPW,), jnp.int32),
        pltpu.VMEM((EPW,), jnp.int32),
        pltpu.VMEM((2, CH, 2 * D), jnp.float32),
        pltpu.VMEM((2, CH, D), jnp.float32),
        pltpu.SemaphoreType.DMA,
        pltpu.SemaphoreType.DMA,
        pltpu.SemaphoreType.DMA,
        pltpu.SemaphoreType.DMA,
        pltpu.SemaphoreType.DMA,
        pltpu.SemaphoreType.DMA,
        pltpu.SemaphoreType.DMA,
        pltpu.SemaphoreType.DMA,
    ],
)
def _gather(ts_hbm, tr_hbm, snd_hbm, rcv_hbm, gs_hbm, gr_hbm,
            idx_s, idx_r, rows_s, rows_r,
            sg0, sg1, sr0, sr1, ws0, ws1, wr0, wr1):
    wid = lax.axis_index("c") * 16 + lax.axis_index("s")
    base = wid * EPW
    pltpu.sync_copy(snd_hbm.at[pl.ds(base, EPW)], idx_s)
    pltpu.sync_copy(rcv_hbm.at[pl.ds(base, EPW)], idx_r)
    sg = (sg0, sg1)
    sr = (sr0, sr1)
    ws = (ws0, ws1)
    wr = (wr0, wr1)

    def body(t, carry):
        g_h, w_h = [], []
        for b in range(2):
            off = (2 * t + b) * CH
            g_h.append(pltpu.async_copy(
                ts_hbm.at[idx_s.at[pl.ds(off, CH)]], rows_s.at[b], sg[b]))
            g_h.append(pltpu.async_copy(
                tr_hbm.at[idx_r.at[pl.ds(off, CH)]], rows_r.at[b], sr[b]))
        for b in range(2):
            off = (2 * t + b) * CH
            g_h[2 * b].wait()
            w_h.append(pltpu.async_copy(
                rows_s.at[b], gs_hbm.at[pl.ds(base + off, CH)], ws[b]))
            g_h[2 * b + 1].wait()
            w_h.append(pltpu.async_copy(
                rows_r.at[b], gr_hbm.at[pl.ds(base + off, CH)], wr[b]))
        for h in w_h:
            h.wait()
        return carry

    lax.fori_loop(0, NCH // 2, body, 0)


def _edge_body(gs_ref, gr_ref, ef_ref, ln_ref, ea_ref,
               w1c_ref, w1d_ref, b1_ref, w2_ref, b2_ref, w3g_ref, wog_ref,
               c_ref):
    a_s = gs_ref[:, :D]
    nf_s = gs_ref[:, D:]
    pre = a_s + gr_ref[...] + b1_ref[...]
    pre = pre + jnp.dot(ef_ref[...], w1c_ref[...], preferred_element_type=jnp.float32)
    pre = pre + ln_ref[...] * w1d_ref[...]
    h1 = pre * jax.nn.sigmoid(pre)
    pre2 = jnp.dot(h1, w2_ref[...], preferred_element_type=jnp.float32) + b2_ref[...]
    h2 = pre2 * jax.nn.sigmoid(pre2)
    tpw = jnp.dot(h2, w3g_ref[...], preferred_element_type=jnp.float32)
    acc = None
    for a in range(DA):
        m = nf_s * tpw[:, a * D:(a + 1) * D] * ea_ref[:, a:a + 1]
        part = jnp.dot(m, wog_ref[a], preferred_element_type=jnp.float32)
        acc = part if acc is None else acc + part
    c_ref[...] = acc


def _edge_compute(gs, gr, ef, ln, ea, w1c, w1d, b1r, w2, b2r, w3g, wog):
    blk = 1024
    grid = E_PAD // blk
    full = lambda i: (0, 0)
    return pl.pallas_call(
        _edge_body,
        grid=(grid,),
        in_specs=[
            pl.BlockSpec((blk, 2 * D), lambda i: (i, 0)),
            pl.BlockSpec((blk, D), lambda i: (i, 0)),
            pl.BlockSpec((blk, DF), lambda i: (i, 0)),
            pl.BlockSpec((blk, 1), lambda i: (i, 0)),
            pl.BlockSpec((blk, DA), lambda i: (i, 0)),
            pl.BlockSpec((DF, D), full),
            pl.BlockSpec((1, D), full),
            pl.BlockSpec((1, D), full),
            pl.BlockSpec((D, D), full),
            pl.BlockSpec((1, D), full),
            pl.BlockSpec((D, MID), full),
            pl.BlockSpec((DA, D, D), lambda i: (0, 0, 0)),
        ],
        out_specs=pl.BlockSpec((blk, D), lambda i: (i, 0)),
        out_shape=jax.ShapeDtypeStruct((E_PAD, D), jnp.float32),
    )(gs, gr, ef, ln, ea, w1c, w1d, b1r, w2, b2r, w3g, wog)


@functools.partial(
    pl.kernel,
    mesh=_mesh,
    out_type=jax.ShapeDtypeStruct((2, 16, RPT, D), jnp.float32),
    scratch_types=[
        pltpu.VMEM((NCH, CH), jnp.int32),
        pltpu.VMEM((2, CH, D), jnp.float32),
        pltpu.VMEM_SHARED((N, D), jnp.float32),
        pltpu.SemaphoreType.DMA,
        pltpu.SemaphoreType.DMA,
    ],
)
def _scatter(c_hbm, rcv3_hbm, zeros_hbm, out_hbm, idx_v, rows_v, accum,
             sl0, sl1):
    c = lax.axis_index("c")
    s = lax.axis_index("s")
    wid = c * 16 + s
    base = wid * EPW
    pltpu.sync_copy(zeros_hbm, accum.at[pl.ds(s * RPT, RPT)])
    pltpu.sync_copy(rcv3_hbm.at[wid], idx_v)
    plsc.subcore_barrier()
    sl = (sl0, sl1)

    def body(t, carry):
        h = []
        for b in range(2):
            j = 2 * t + b
            h.append(pltpu.async_copy(
                c_hbm.at[pl.ds(base + j * CH, CH)], rows_v.at[b], sl[b]))
        for b in range(2):
            j = 2 * t + b
            h[b].wait()
            pltpu.sync_copy(rows_v.at[b], accum.at[idx_v.at[j]], add=True)
        return carry

    lax.fori_loop(0, NCH // 2, body, 0)
    plsc.subcore_barrier()
    pltpu.sync_copy(accum.at[pl.ds(s * RPT, RPT)], out_hbm.at[c, s])


def _add_body(p0_ref, p1_ref, o_ref):
    o_ref[...] = p0_ref[...] + p1_ref[...]


def _final_add(p0, p1):
    blk = 1000
    return pl.pallas_call(
        _add_body,
        grid=(N // blk,),
        in_specs=[
            pl.BlockSpec((blk, D), lambda i: (i, 0)),
            pl.BlockSpec((blk, D), lambda i: (i, 0)),
        ],
        out_specs=pl.BlockSpec((blk, D), lambda i: (i, 0)),
        out_shape=jax.ShapeDtypeStruct((N, D), jnp.float32),
    )(p0, p1)


def kernel(node_feats, edge_attrs, edge_feats, lengths, edge_index,
           W_scalar, W_up, W1, b1, W2, b2, W3, W_out):
    pad = E_PAD - E_IN
    snd = jnp.pad(edge_index[0], (0, pad))
    rcv = jnp.pad(edge_index[1], (0, pad))
    ea = jnp.pad(edge_attrs, ((0, pad), (0, 0)))
    ef = jnp.pad(edge_feats, ((0, pad), (0, 0)))
    ln = jnp.pad(lengths, ((0, pad), (0, 0)))

    w1a = W1[:D]
    w1b = W1[D:2 * D]
    w1c = W1[2 * D:2 * D + DF]
    w1d = W1[2 * D + DF:2 * D + DF + 1]
    b1r = b1.reshape(1, D)
    b2r = b2.reshape(1, D)
    w3g = W3.reshape(D, D, DA).transpose(0, 2, 1).reshape(D, MID)
    wog = W_out.reshape(D, DA, D).transpose(1, 0, 2) / AVG

    ts, tr = _node_precompute(node_feats, W_scalar, W_up, w1a, w1b)
    gs, gr = _gather(ts, tr, snd, rcv)
    cmsg = _edge_compute(gs, gr, ef, ln, ea, w1c, w1d, b1r, W2, b2r, w3g, wog)
    partials = _scatter(cmsg, rcv.reshape(NW, NCH, CH),
                        jnp.zeros((RPT, D), jnp.float32)).reshape(2, N, D)
    out = _final_add(partials[0], partials[1])
    return out.reshape(N, D, 1)
